# baseline (speedup 1.0000x reference)
import jax
import jax.numpy as jnp
from jax import lax
from jax.experimental import pallas as pl
from jax.experimental.pallas import tpu as pltpu
from jax.experimental.pallas import tpu_sc as plsc
import functools

N = 10000
D = 256
NV = D // 16
S = 64
W = 160
NG = N // 16
NC = 2
NS = 16
NW = NC * NS
SPT = S // NW


def _sc_pool_body(x_hbm, b_hbm, out_hbm, bat_v, xw_v, out_v, sems):
    core = lax.axis_index("c")
    sid = lax.axis_index("s")
    wid = sid * NC + core
    s0 = wid * SPT

    pltpu.sync_copy(b_hbm, bat_v)

    thr = [jnp.full((16,), s0 + t, jnp.int32) for t in range(SPT + 1)]
    zi = jnp.zeros((16,), jnp.int32)

    def count_body(g5, accs):
        for u in range(5):
            v = bat_v[pl.ds((g5 * 5 + u) * 16, 16)]
            accs = tuple(a + jnp.minimum(jnp.maximum(t - v, 0), 1)
                         for a, t in zip(accs, thr))
        return accs

    accs = lax.fori_loop(0, NG // 5, count_body, (zi,) * (SPT + 1))

    def _hsum(a):
        t = a[0]
        for l in range(1, 16):
            t = t + a[l]
        return t

    cuts = [_hsum(a) for a in accs]

    zeros16 = jnp.zeros((16,), jnp.float32)
    for k in range(SPT):
        lo_row, hi_row = cuts[k], cuts[k + 1]
        num = hi_row - lo_row
        a0 = lo_row & ~7
        nwin = ((hi_row - a0 + (W - 1)) // W) * jnp.minimum(num, 1)

        def wbase(i):
            return pl.multiple_of(jnp.minimum(a0 + i * W, N - W), 8)

        @pl.when(num > 0)
        def _():
            pltpu.async_copy(x_hbm.at[pl.ds(wbase(0), W)], xw_v.at[0],
                             sems.at[0])

        def win_body(i, sums):
            buf = i & 1
            start = a0 + i * W
            base = wbase(i)
            pltpu.make_async_copy(x_hbm.at[pl.ds(base, W)],
                                  xw_v.at[buf], sems.at[buf]).wait()

            @pl.when(i + 1 < nwin)
            def _():
                pltpu.async_copy(x_hbm.at[pl.ds(wbase(i + 1), W)],
                                 xw_v.at[(i + 1) & 1], sems.at[(i + 1) & 1])

            lo_l = jnp.maximum(lo_row, start) - base
            hi_l = jnp.minimum(hi_row, start + W) - base

            def row_body(rl, sums2):
                vals = [xw_v[buf, rl, pl.ds(kk * 16, 16)]
                        for kk in range(NV)]
                return tuple(a + b for a, b in zip(sums2, vals))

            return lax.fori_loop(lo_l, hi_l, row_body, sums)

        sums = lax.fori_loop(0, nwin, win_body, (zeros16,) * NV)
        cnt16 = jnp.full((16,), jnp.maximum(num, 1),
                         jnp.int32).astype(jnp.float32)
        scale16 = jnp.full((16,), 1.0, jnp.float32) / cnt16
        for kk in range(NV):
            out_v[k, pl.ds(kk * 16, 16)] = sums[kk] * scale16

    pltpu.sync_copy(out_v, out_hbm.at[pl.ds(s0, SPT)])


_sc_pool = functools.partial(
    pl.kernel,
    out_type=[
        jax.ShapeDtypeStruct((S, D), jnp.float32),
    ],
    mesh=plsc.VectorSubcoreMesh(core_axis_name="c", subcore_axis_name="s"),
    scratch_types=[
        pltpu.VMEM((N,), jnp.int32),
        pltpu.VMEM((2, W, D), jnp.float32),
        pltpu.VMEM((SPT, D), jnp.float32),
        pltpu.SemaphoreType.DMA((2,)),
    ],
)(_sc_pool_body)


@jax.jit
def kernel(x, edge_index, batch):
    del edge_index
    (out,) = _sc_pool(x, batch.astype(jnp.int32))
    return out
